# Initial kernel scaffold; baseline (speedup 1.0000x reference)
import jax
import jax.numpy as jnp
from jax.experimental import pallas as pl


def kernel(A_g, X_g, U_g, S_g, batch, W1x, b1x, W2x, b2x, eps_x, gamma_x, beta_x, W1u, b1u, W2u, b2u, eps_u, gamma_u, beta_u, Wz, bz, Wa1, ba1, Wa2, ba2, noise_gx, noise_bx, noise_gu, noise_bu):
    raise NotImplementedError("write your pallas kernel here")


# speedup vs baseline: 4.4169x; 4.4169x over previous
import functools

import jax
import jax.numpy as jnp
from jax import lax
from jax.experimental import pallas as pl
from jax.experimental.pallas import tpu as pltpu
from jax.experimental.pallas import tpu_sc as plsc

N = 10000
E = 320000
D = 128
B = 256

NPAD = 10240
STRIPE = NPAD // 16
CHUNK = 128
CH_PER_TILE = 160
EPT = CH_PER_TILE * CHUNK
EPAD = EPT * 16


def _sc_aggregate(src2d, dst2d, x_g, u_g, zinit):
    mesh = plsc.VectorSubcoreMesh(core_axis_name="c", subcore_axis_name="s")

    @functools.partial(
        pl.kernel,
        mesh=mesh,
        out_type=[jax.ShapeDtypeStruct((NPAD, D), jnp.float32),
                  jax.ShapeDtypeStruct((NPAD, D), jnp.float32)],
        scratch_types=[
            pltpu.VMEM((CHUNK,), jnp.int32),
            pltpu.VMEM((CHUNK,), jnp.int32),
            pltpu.VMEM((CHUNK, D), jnp.float32),
            pltpu.VMEM_SHARED((NPAD, D), jnp.float32),
            pltpu.SemaphoreType.DMA,
        ],
    )
    def sc_kernel(src_hbm, dst_hbm, x_hbm, u_hbm, z_hbm, outx_hbm, outu_hbm,
                  src_v, dst_v, rows_v, agg_sh, sem):
        cid = lax.axis_index("c")
        sid = lax.axis_index("s")

        def work(table_hbm, out_hbm):
            pltpu.sync_copy(z_hbm.at[pl.ds(sid * STRIPE, STRIPE)],
                            agg_sh.at[pl.ds(sid * STRIPE, STRIPE)])
            plsc.subcore_barrier()

            def body(i, carry):
                base = pl.multiple_of(sid * EPT + i * CHUNK, CHUNK)
                pltpu.sync_copy(src_hbm.at[pl.ds(base, CHUNK)], src_v)
                pltpu.sync_copy(dst_hbm.at[pl.ds(base, CHUNK)], dst_v)
                pltpu.async_copy(table_hbm.at[src_v], rows_v, sem).wait()
                pltpu.sync_copy(rows_v, agg_sh.at[dst_v], add=True)
                return carry

            lax.fori_loop(0, CH_PER_TILE, body, 0)
            plsc.subcore_barrier()
            pltpu.sync_copy(agg_sh.at[pl.ds(sid * STRIPE, STRIPE)],
                            out_hbm.at[pl.ds(sid * STRIPE, STRIPE)])

        @pl.when(cid == 0)
        def _():
            work(x_hbm, outx_hbm)

        @pl.when(cid == 1)
        def _():
            work(u_hbm, outu_hbm)

    return sc_kernel(src2d, dst2d, x_g, u_g, zinit)


def _softplus(x):
    return jnp.logaddexp(x, 0.0)


_DN = (((1,), (1,)), ((), ()))
_HI = lax.Precision.HIGHEST

NBLK = 10
BLK = N // NBLK


def _tc_body(x_ref, u_ref, ax_ref, au_ref, b3_ref, sg_ref,
             w1x_ref, b1x_ref, w2x_ref, b2x_ref, epsx_ref, gx_ref, btx_ref, ngx_ref, nbx_ref,
             w1u_ref, b1u_ref, w2u_ref, b2u_ref, epsu_ref, gu_ref, btu_ref, ngu_ref, nbu_ref,
             wz_ref, bz_ref, wa1x_ref, wa1u_ref, wa1z_ref, ba1_ref, wa2_ref, ba2_ref,
             out_ref, accx, accu, cnt):
    i = pl.program_id(0)

    def encoder(xb, ab, w1, b1, w2, b2, eps, gam, bet, ng, nb):
        h0 = (1.0 + eps) * xb + ab
        z = lax.dot_general(h0, w1, _DN, precision=_HI,
                            preferred_element_type=jnp.float32) + b1
        z = jnp.maximum(z, 0.0)
        z = lax.dot_general(z, w2, _DN, precision=_HI,
                            preferred_element_type=jnp.float32) + b2
        gs = 1.0 + _softplus(gam) * ng
        bs = _softplus(bet) * nb
        z = gs * z + bs
        return z * jax.nn.sigmoid(z)

    hx = encoder(x_ref[...], ax_ref[...], w1x_ref[...], b1x_ref[...],
                 w2x_ref[...], b2x_ref[...], epsx_ref[0, 0], gx_ref[...],
                 btx_ref[...], ngx_ref[...], nbx_ref[...])
    hu = encoder(u_ref[...], au_ref[...], w1u_ref[...], b1u_ref[...],
                 w2u_ref[...], b2u_ref[...], epsu_ref[0, 0], gu_ref[...],
                 btu_ref[...], ngu_ref[...], nbu_ref[...])

    row = jnp.reshape(b3_ref[...], (1, BLK))
    iot = lax.broadcasted_iota(jnp.int32, (B, BLK), 0)
    oh = (iot == row).astype(jnp.float32)
    px = lax.dot_general(oh, hx, (((1,), (0,)), ((), ())), precision=_HI,
                         preferred_element_type=jnp.float32)
    pu = lax.dot_general(oh, hu, (((1,), (0,)), ((), ())), precision=_HI,
                         preferred_element_type=jnp.float32)
    pc = lax.dot_general(oh, jnp.ones((BLK, D), jnp.float32),
                         (((1,), (0,)), ((), ())), precision=_HI,
                         preferred_element_type=jnp.float32)

    @pl.when(i == 0)
    def _():
        accx[...] = px
        accu[...] = pu
        cnt[...] = pc

    @pl.when(i > 0)
    def _():
        accx[...] += px
        accu[...] += pu
        cnt[...] += pc

    @pl.when(i == NBLK - 1)
    def _():
        c = jnp.maximum(cnt[...], 1.0)
        hxm = accx[...] / c
        hum = accu[...] / c
        hz = lax.dot_general(sg_ref[...], wz_ref[...], _DN, precision=_HI,
                             preferred_element_type=jnp.float32) + bz_ref[...]
        s = (lax.dot_general(hxm, wa1x_ref[...], _DN, precision=_HI,
                             preferred_element_type=jnp.float32)
             + lax.dot_general(hum, wa1u_ref[...], _DN, precision=_HI,
                               preferred_element_type=jnp.float32)
             + lax.dot_general(hz, wa1z_ref[...], _DN, precision=_HI,
                               preferred_element_type=jnp.float32)
             + ba1_ref[...])
        s = jnp.maximum(s, 0.0)
        a0 = jnp.sum(s * wa2_ref[0:1, :], axis=1, keepdims=True) + ba2_ref[0, 0]
        a1 = jnp.sum(s * wa2_ref[1:2, :], axis=1, keepdims=True) + ba2_ref[0, 1]
        a2 = jnp.sum(s * wa2_ref[2:3, :], axis=1, keepdims=True) + ba2_ref[0, 2]
        m = jnp.maximum(jnp.maximum(a0, a1), a2)
        e0 = jnp.exp(a0 - m)
        e1 = jnp.exp(a1 - m)
        e2 = jnp.exp(a2 - m)
        out_ref[...] = (e0 * hxm + e1 * hum + e2 * hz) / (e0 + e1 + e2)


def _tc_dense(x_g, u_g, aggx, aggu, batch3, s_g, *weights):
    full = lambda shape: pl.BlockSpec(shape, lambda i: (0,) * len(shape))
    blk = pl.BlockSpec((BLK, D), lambda i: (i, 0))
    in_specs = [
        blk, blk, blk, blk,
        pl.BlockSpec((1, 1, BLK), lambda i: (i, 0, 0)),
        full((B, D)),
        full((D, D)), full((1, D)), full((D, D)), full((1, D)),
        full((1, 1)), full((1, D)), full((1, D)), full((1, D)), full((1, D)),
        full((D, D)), full((1, D)), full((D, D)), full((1, D)),
        full((1, 1)), full((1, D)), full((1, D)), full((1, D)), full((1, D)),
        full((D, D)), full((1, D)),
        full((D, D)), full((D, D)), full((D, D)), full((1, D)),
        full((8, D)), full((1, D)),
    ]
    return pl.pallas_call(
        _tc_body,
        grid=(NBLK,),
        in_specs=in_specs,
        out_specs=pl.BlockSpec((B, D), lambda i: (0, 0)),
        out_shape=jax.ShapeDtypeStruct((B, D), jnp.float32),
        scratch_shapes=[
            pltpu.VMEM((B, D), jnp.float32),
            pltpu.VMEM((B, D), jnp.float32),
            pltpu.VMEM((B, D), jnp.float32),
        ],
        compiler_params=pltpu.CompilerParams(
            dimension_semantics=("arbitrary",)),
    )(x_g, u_g, aggx, aggu, batch3, s_g, *weights)


def kernel(A_g, X_g, U_g, S_g, batch,
           W1x, b1x, W2x, b2x, eps_x, gamma_x, beta_x,
           W1u, b1u, W2u, b2u, eps_u, gamma_u, beta_u,
           Wz, bz, Wa1, ba1, Wa2, ba2,
           noise_gx, noise_bx, noise_gu, noise_bu):
    npad_e = EPAD - E
    pad_src = (jnp.arange(npad_e, dtype=jnp.int32) % 128)
    pad_dst = N + (jnp.arange(npad_e, dtype=jnp.int32) % (NPAD - N))
    src2d = jnp.concatenate([A_g[0].astype(jnp.int32), pad_src])
    dst2d = jnp.concatenate([A_g[1].astype(jnp.int32), pad_dst])
    zinit = jnp.zeros((NPAD, D), jnp.float32)

    aggx, aggu = _sc_aggregate(src2d, dst2d, X_g, U_g, zinit)

    batch3 = batch.astype(jnp.int32).reshape(NBLK, 1, BLK)
    r = lambda a: a.reshape(1, D)
    weights = (
        W1x, r(b1x), W2x, r(b2x), eps_x.reshape(1, 1), r(gamma_x), r(beta_x),
        noise_gx, noise_bx,
        W1u, r(b1u), W2u, r(b2u), eps_u.reshape(1, 1), r(gamma_u), r(beta_u),
        noise_gu, noise_bu,
        Wz, r(bz),
        Wa1[:, 0:D], Wa1[:, D:2 * D], Wa1[:, 2 * D:3 * D], r(ba1),
        jnp.zeros((8, D), jnp.float32).at[0:3].set(Wa2),
        jnp.zeros((1, D), jnp.float32).at[0, 0:3].set(ba2),
    )
    return _tc_dense(X_g, U_g, aggx[:N], aggu[:N], batch3, S_g, *weights)


# speedup vs baseline: 7.9052x; 1.7898x over previous
import functools

import jax
import jax.numpy as jnp
from jax import lax
from jax.experimental import pallas as pl
from jax.experimental.pallas import tpu as pltpu
from jax.experimental.pallas import tpu_sc as plsc

N = 10000
E = 320000
D = 128
B = 256

NPAD = 10240
STRIPE = NPAD // 16
CHUNK = 128
CH_PER_TILE = 160
GRP = 16
NGRP = CH_PER_TILE // GRP
EPT = CH_PER_TILE * CHUNK
EPAD = EPT * 16


def _sc_aggregate(src2d, dst2d, x_g, u_g, zinit):
    mesh = plsc.VectorSubcoreMesh(core_axis_name="c", subcore_axis_name="s")

    @functools.partial(
        pl.kernel,
        mesh=mesh,
        out_type=[jax.ShapeDtypeStruct((NPAD, D), jnp.float32),
                  jax.ShapeDtypeStruct((NPAD, D), jnp.float32)],
        scratch_types=[
            pltpu.VMEM((GRP, CHUNK), jnp.int32),
            pltpu.VMEM((GRP, CHUNK), jnp.int32),
            pltpu.VMEM((CHUNK, D), jnp.float32),
            pltpu.VMEM((CHUNK, D), jnp.float32),
            pltpu.VMEM_SHARED((NPAD, D), jnp.float32),
            pltpu.SemaphoreType.DMA,
            pltpu.SemaphoreType.DMA,
            pltpu.SemaphoreType.DMA,
            pltpu.SemaphoreType.DMA,
        ],
    )
    def sc_kernel(src_hbm, dst_hbm, x_hbm, u_hbm, z_hbm, outx_hbm, outu_hbm,
                  src_v, dst_v, rows0, rows1, agg_sh,
                  sem_g0, sem_g1, sem_s0, sem_s1):
        cid = lax.axis_index("c")
        sid = lax.axis_index("s")

        def work(table_hbm, out_hbm):
            rows = (rows0, rows1)
            sem_g = (sem_g0, sem_g1)
            sem_s = (sem_s0, sem_s1)
            pltpu.sync_copy(z_hbm.at[pl.ds(sid * STRIPE, STRIPE)],
                            agg_sh.at[pl.ds(sid * STRIPE, STRIPE)])
            plsc.subcore_barrier()

            def group(g, carry):
                row_base = sid * CH_PER_TILE + g * GRP
                pltpu.sync_copy(src_hbm.at[pl.ds(row_base, GRP)], src_v)
                pltpu.sync_copy(dst_hbm.at[pl.ds(row_base, GRP)], dst_v)
                scat = [None, None]
                gath = [None, None]
                gath[0] = pltpu.async_copy(table_hbm.at[src_v.at[0]],
                                           rows[0], sem_g[0])
                for b in range(GRP):
                    cur, nxt = b % 2, (b + 1) % 2
                    if b + 1 < GRP:
                        if scat[nxt] is not None:
                            scat[nxt].wait()
                        gath[nxt] = pltpu.async_copy(
                            table_hbm.at[src_v.at[b + 1]], rows[nxt], sem_g[nxt])
                    gath[cur].wait()
                    scat[cur] = pltpu.async_copy(
                        rows[cur], agg_sh.at[dst_v.at[b]], sem_s[cur], add=True)
                scat[0].wait()
                scat[1].wait()
                return carry

            lax.fori_loop(0, NGRP, group, 0)
            plsc.subcore_barrier()
            pltpu.sync_copy(agg_sh.at[pl.ds(sid * STRIPE, STRIPE)],
                            out_hbm.at[pl.ds(sid * STRIPE, STRIPE)])

        @pl.when(cid == 0)
        def _():
            work(x_hbm, outx_hbm)

        @pl.when(cid == 1)
        def _():
            work(u_hbm, outu_hbm)

    return sc_kernel(src2d, dst2d, x_g, u_g, zinit)


def _softplus(x):
    return jnp.logaddexp(x, 0.0)


_DN = (((1,), (1,)), ((), ()))
_HI = lax.Precision.HIGHEST

NBLK = 10
BLK = N // NBLK


def _tc_body(x_ref, u_ref, ax_ref, au_ref, b3_ref, sg_ref,
             w1x_ref, b1x_ref, w2x_ref, b2x_ref, epsx_ref, gx_ref, btx_ref, ngx_ref, nbx_ref,
             w1u_ref, b1u_ref, w2u_ref, b2u_ref, epsu_ref, gu_ref, btu_ref, ngu_ref, nbu_ref,
             wz_ref, bz_ref, wa1x_ref, wa1u_ref, wa1z_ref, ba1_ref, wa2_ref, ba2_ref,
             out_ref, accx, accu, cnt):
    i = pl.program_id(0)

    def encoder(xb, ab, w1, b1, w2, b2, eps, gam, bet, ng, nb):
        h0 = (1.0 + eps) * xb + ab
        z = lax.dot_general(h0, w1, _DN, precision=_HI,
                            preferred_element_type=jnp.float32) + b1
        z = jnp.maximum(z, 0.0)
        z = lax.dot_general(z, w2, _DN, precision=_HI,
                            preferred_element_type=jnp.float32) + b2
        gs = 1.0 + _softplus(gam) * ng
        bs = _softplus(bet) * nb
        z = gs * z + bs
        return z * jax.nn.sigmoid(z)

    hx = encoder(x_ref[...], ax_ref[...], w1x_ref[...], b1x_ref[...],
                 w2x_ref[...], b2x_ref[...], epsx_ref[0, 0], gx_ref[...],
                 btx_ref[...], ngx_ref[...], nbx_ref[...])
    hu = encoder(u_ref[...], au_ref[...], w1u_ref[...], b1u_ref[...],
                 w2u_ref[...], b2u_ref[...], epsu_ref[0, 0], gu_ref[...],
                 btu_ref[...], ngu_ref[...], nbu_ref[...])

    row = jnp.reshape(b3_ref[...], (1, BLK))
    iot = lax.broadcasted_iota(jnp.int32, (B, BLK), 0)
    oh = (iot == row).astype(jnp.float32)
    px = lax.dot_general(oh, hx, (((1,), (0,)), ((), ())), precision=_HI,
                         preferred_element_type=jnp.float32)
    pu = lax.dot_general(oh, hu, (((1,), (0,)), ((), ())), precision=_HI,
                         preferred_element_type=jnp.float32)
    pc = lax.dot_general(oh, jnp.ones((BLK, D), jnp.float32),
                         (((1,), (0,)), ((), ())), precision=_HI,
                         preferred_element_type=jnp.float32)

    @pl.when(i == 0)
    def _():
        accx[...] = px
        accu[...] = pu
        cnt[...] = pc

    @pl.when(i > 0)
    def _():
        accx[...] += px
        accu[...] += pu
        cnt[...] += pc

    @pl.when(i == NBLK - 1)
    def _():
        c = jnp.maximum(cnt[...], 1.0)
        hxm = accx[...] / c
        hum = accu[...] / c
        hz = lax.dot_general(sg_ref[...], wz_ref[...], _DN, precision=_HI,
                             preferred_element_type=jnp.float32) + bz_ref[...]
        s = (lax.dot_general(hxm, wa1x_ref[...], _DN, precision=_HI,
                             preferred_element_type=jnp.float32)
             + lax.dot_general(hum, wa1u_ref[...], _DN, precision=_HI,
                               preferred_element_type=jnp.float32)
             + lax.dot_general(hz, wa1z_ref[...], _DN, precision=_HI,
                               preferred_element_type=jnp.float32)
             + ba1_ref[...])
        s = jnp.maximum(s, 0.0)
        a0 = jnp.sum(s * wa2_ref[0:1, :], axis=1, keepdims=True) + ba2_ref[0, 0]
        a1 = jnp.sum(s * wa2_ref[1:2, :], axis=1, keepdims=True) + ba2_ref[0, 1]
        a2 = jnp.sum(s * wa2_ref[2:3, :], axis=1, keepdims=True) + ba2_ref[0, 2]
        m = jnp.maximum(jnp.maximum(a0, a1), a2)
        e0 = jnp.exp(a0 - m)
        e1 = jnp.exp(a1 - m)
        e2 = jnp.exp(a2 - m)
        out_ref[...] = (e0 * hxm + e1 * hum + e2 * hz) / (e0 + e1 + e2)


def _tc_dense(x_g, u_g, aggx, aggu, batch3, s_g, *weights):
    full = lambda shape: pl.BlockSpec(shape, lambda i: (0,) * len(shape))
    blk = pl.BlockSpec((BLK, D), lambda i: (i, 0))
    in_specs = [
        blk, blk, blk, blk,
        pl.BlockSpec((1, 1, BLK), lambda i: (i, 0, 0)),
        full((B, D)),
        full((D, D)), full((1, D)), full((D, D)), full((1, D)),
        full((1, 1)), full((1, D)), full((1, D)), full((1, D)), full((1, D)),
        full((D, D)), full((1, D)), full((D, D)), full((1, D)),
        full((1, 1)), full((1, D)), full((1, D)), full((1, D)), full((1, D)),
        full((D, D)), full((1, D)),
        full((D, D)), full((D, D)), full((D, D)), full((1, D)),
        full((8, D)), full((1, D)),
    ]
    return pl.pallas_call(
        _tc_body,
        grid=(NBLK,),
        in_specs=in_specs,
        out_specs=pl.BlockSpec((B, D), lambda i: (0, 0)),
        out_shape=jax.ShapeDtypeStruct((B, D), jnp.float32),
        scratch_shapes=[
            pltpu.VMEM((B, D), jnp.float32),
            pltpu.VMEM((B, D), jnp.float32),
            pltpu.VMEM((B, D), jnp.float32),
        ],
        compiler_params=pltpu.CompilerParams(
            dimension_semantics=("arbitrary",)),
    )(x_g, u_g, aggx, aggu, batch3, s_g, *weights)


def kernel(A_g, X_g, U_g, S_g, batch,
           W1x, b1x, W2x, b2x, eps_x, gamma_x, beta_x,
           W1u, b1u, W2u, b2u, eps_u, gamma_u, beta_u,
           Wz, bz, Wa1, ba1, Wa2, ba2,
           noise_gx, noise_bx, noise_gu, noise_bu):
    npad_e = EPAD - E
    pad_src = (jnp.arange(npad_e, dtype=jnp.int32) % 128)
    pad_dst = N + (jnp.arange(npad_e, dtype=jnp.int32) % (NPAD - N))
    src2d = jnp.concatenate([A_g[0].astype(jnp.int32), pad_src]).reshape(-1, CHUNK)
    dst2d = jnp.concatenate([A_g[1].astype(jnp.int32), pad_dst]).reshape(-1, CHUNK)
    zinit = jnp.zeros((NPAD, D), jnp.float32)

    aggx, aggu = _sc_aggregate(src2d, dst2d, X_g, U_g, zinit)

    batch3 = batch.astype(jnp.int32).reshape(NBLK, 1, BLK)
    r = lambda a: a.reshape(1, D)
    weights = (
        W1x, r(b1x), W2x, r(b2x), eps_x.reshape(1, 1), r(gamma_x), r(beta_x),
        noise_gx, noise_bx,
        W1u, r(b1u), W2u, r(b2u), eps_u.reshape(1, 1), r(gamma_u), r(beta_u),
        noise_gu, noise_bu,
        Wz, r(bz),
        Wa1[:, 0:D], Wa1[:, D:2 * D], Wa1[:, 2 * D:3 * D], r(ba1),
        jnp.zeros((8, D), jnp.float32).at[0:3].set(Wa2),
        jnp.zeros((1, D), jnp.float32).at[0, 0:3].set(ba2),
    )
    return _tc_dense(X_g, U_g, aggx[:N], aggu[:N], batch3, S_g, *weights)


# speedup vs baseline: 8.1553x; 1.0316x over previous
import functools

import jax
import jax.numpy as jnp
from jax import lax
from jax.experimental import pallas as pl
from jax.experimental.pallas import tpu as pltpu
from jax.experimental.pallas import tpu_sc as plsc

N = 10000
E = 320000
D = 128
B = 256

NPAD = 10240
STRIPE = NPAD // 16
CHUNK = 128
CH_PER_TILE = 160
GRP = 16
NGRP = CH_PER_TILE // GRP
EPT = CH_PER_TILE * CHUNK
EPAD = EPT * 16


def _sc_aggregate(src2d, dst2d, x_g, u_g, zinit):
    mesh = plsc.VectorSubcoreMesh(core_axis_name="c", subcore_axis_name="s")

    @functools.partial(
        pl.kernel,
        mesh=mesh,
        out_type=[jax.ShapeDtypeStruct((NPAD, D), jnp.float32),
                  jax.ShapeDtypeStruct((NPAD, D), jnp.float32)],
        scratch_types=[
            pltpu.VMEM((GRP, CHUNK), jnp.int32),
            pltpu.VMEM((GRP, CHUNK), jnp.int32),
            pltpu.VMEM((GRP, CHUNK), jnp.int32),
            pltpu.VMEM((GRP, CHUNK), jnp.int32),
            pltpu.VMEM((CHUNK, D), jnp.float32),
            pltpu.VMEM((CHUNK, D), jnp.float32),
            pltpu.VMEM_SHARED((NPAD, D), jnp.float32),
            pltpu.SemaphoreType.DMA,
            pltpu.SemaphoreType.DMA,
            pltpu.SemaphoreType.DMA,
            pltpu.SemaphoreType.DMA,
            pltpu.SemaphoreType.DMA,
            pltpu.SemaphoreType.DMA,
        ],
    )
    def sc_kernel(src_hbm, dst_hbm, x_hbm, u_hbm, z_hbm, outx_hbm, outu_hbm,
                  src_v0, src_v1, dst_v0, dst_v1, rows0, rows1, agg_sh,
                  sem_g0, sem_g1, sem_s0, sem_s1, sem_i0, sem_i1):
        cid = lax.axis_index("c")
        sid = lax.axis_index("s")

        def work(table_hbm, out_hbm):
            rows = (rows0, rows1)
            sem_g = (sem_g0, sem_g1)
            sem_s = (sem_s0, sem_s1)
            src_v = (src_v0, src_v1)
            dst_v = (dst_v0, dst_v1)
            sem_i = (sem_i0, sem_i1)

            def idx_load(g, s):
                row_base = sid * CH_PER_TILE + g * GRP
                a = pltpu.async_copy(src_hbm.at[pl.ds(row_base, GRP)],
                                     src_v[s], sem_i[s])
                b = pltpu.async_copy(dst_hbm.at[pl.ds(row_base, GRP)],
                                     dst_v[s], sem_i[s])
                return (a, b)

            def idx_wait_desc(g, s):
                row_base = sid * CH_PER_TILE + g * GRP
                a = pltpu.make_async_copy(src_hbm.at[pl.ds(row_base, GRP)],
                                          src_v[s], sem_i[s])
                b = pltpu.make_async_copy(dst_hbm.at[pl.ds(row_base, GRP)],
                                          dst_v[s], sem_i[s])
                return (a, b)

            idx_load(0, 0)
            pltpu.sync_copy(z_hbm.at[pl.ds(sid * STRIPE, STRIPE)],
                            agg_sh.at[pl.ds(sid * STRIPE, STRIPE)])
            plsc.subcore_barrier()

            def run_group(g, s, idx_desc):
                for dsc in idx_desc:
                    dsc.wait()
                nxt_idx = idx_load(g + 1, 1 - s)
                scat = [None, None]
                gath = [None, None]
                gath[0] = pltpu.async_copy(table_hbm.at[src_v[s].at[0]],
                                           rows[0], sem_g[0])
                for b in range(GRP):
                    cur, nxt = b % 2, (b + 1) % 2
                    if b + 1 < GRP:
                        if scat[nxt] is not None:
                            scat[nxt].wait()
                        gath[nxt] = pltpu.async_copy(
                            table_hbm.at[src_v[s].at[b + 1]], rows[nxt],
                            sem_g[nxt])
                    gath[cur].wait()
                    scat[cur] = pltpu.async_copy(
                        rows[cur], agg_sh.at[dst_v[s].at[b]], sem_s[cur],
                        add=True)
                scat[0].wait()
                scat[1].wait()
                return nxt_idx

            def super_group(sg, carry):
                nxt = run_group(2 * sg, 0, idx_wait_desc(2 * sg, 0))
                run_group(2 * sg + 1, 1, nxt)
                return carry

            lax.fori_loop(0, NGRP // 2, super_group, 0)
            for dsc in idx_wait_desc(NGRP, 0):
                dsc.wait()
            plsc.subcore_barrier()
            pltpu.sync_copy(agg_sh.at[pl.ds(sid * STRIPE, STRIPE)],
                            out_hbm.at[pl.ds(sid * STRIPE, STRIPE)])

        @pl.when(cid == 0)
        def _():
            work(x_hbm, outx_hbm)

        @pl.when(cid == 1)
        def _():
            work(u_hbm, outu_hbm)

    return sc_kernel(src2d, dst2d, x_g, u_g, zinit)


def _softplus(x):
    return jnp.logaddexp(x, 0.0)


_DN = (((1,), (1,)), ((), ()))
_HI = lax.Precision.HIGHEST

NBLK = 10
BLK = N // NBLK


def _tc_body(x_ref, u_ref, ax_ref, au_ref, b3_ref, sg_ref,
             w1x_ref, b1x_ref, w2x_ref, b2x_ref, epsx_ref, gx_ref, btx_ref, ngx_ref, nbx_ref,
             w1u_ref, b1u_ref, w2u_ref, b2u_ref, epsu_ref, gu_ref, btu_ref, ngu_ref, nbu_ref,
             wz_ref, bz_ref, wa1x_ref, wa1u_ref, wa1z_ref, ba1_ref, wa2_ref, ba2_ref,
             out_ref, accx, accu, cnt):
    i = pl.program_id(0)

    def encoder(xb, ab, w1, b1, w2, b2, eps, gam, bet, ng, nb):
        h0 = (1.0 + eps) * xb + ab
        z = lax.dot_general(h0, w1, _DN, precision=_HI,
                            preferred_element_type=jnp.float32) + b1
        z = jnp.maximum(z, 0.0)
        z = lax.dot_general(z, w2, _DN, precision=_HI,
                            preferred_element_type=jnp.float32) + b2
        gs = 1.0 + _softplus(gam) * ng
        bs = _softplus(bet) * nb
        z = gs * z + bs
        return z * jax.nn.sigmoid(z)

    hx = encoder(x_ref[...], ax_ref[...], w1x_ref[...], b1x_ref[...],
                 w2x_ref[...], b2x_ref[...], epsx_ref[0, 0], gx_ref[...],
                 btx_ref[...], ngx_ref[...], nbx_ref[...])
    hu = encoder(u_ref[...], au_ref[...], w1u_ref[...], b1u_ref[...],
                 w2u_ref[...], b2u_ref[...], epsu_ref[0, 0], gu_ref[...],
                 btu_ref[...], ngu_ref[...], nbu_ref[...])

    row = jnp.reshape(b3_ref[...], (1, BLK))
    iot = lax.broadcasted_iota(jnp.int32, (B, BLK), 0)
    oh = (iot == row).astype(jnp.float32)
    px = lax.dot_general(oh, hx, (((1,), (0,)), ((), ())), precision=_HI,
                         preferred_element_type=jnp.float32)
    pu = lax.dot_general(oh, hu, (((1,), (0,)), ((), ())), precision=_HI,
                         preferred_element_type=jnp.float32)
    pc = lax.dot_general(oh, jnp.ones((BLK, D), jnp.float32),
                         (((1,), (0,)), ((), ())), precision=_HI,
                         preferred_element_type=jnp.float32)

    @pl.when(i == 0)
    def _():
        accx[...] = px
        accu[...] = pu
        cnt[...] = pc

    @pl.when(i > 0)
    def _():
        accx[...] += px
        accu[...] += pu
        cnt[...] += pc

    @pl.when(i == NBLK - 1)
    def _():
        c = jnp.maximum(cnt[...], 1.0)
        hxm = accx[...] / c
        hum = accu[...] / c
        hz = lax.dot_general(sg_ref[...], wz_ref[...], _DN, precision=_HI,
                             preferred_element_type=jnp.float32) + bz_ref[...]
        s = (lax.dot_general(hxm, wa1x_ref[...], _DN, precision=_HI,
                             preferred_element_type=jnp.float32)
             + lax.dot_general(hum, wa1u_ref[...], _DN, precision=_HI,
                               preferred_element_type=jnp.float32)
             + lax.dot_general(hz, wa1z_ref[...], _DN, precision=_HI,
                               preferred_element_type=jnp.float32)
             + ba1_ref[...])
        s = jnp.maximum(s, 0.0)
        a0 = jnp.sum(s * wa2_ref[0:1, :], axis=1, keepdims=True) + ba2_ref[0, 0]
        a1 = jnp.sum(s * wa2_ref[1:2, :], axis=1, keepdims=True) + ba2_ref[0, 1]
        a2 = jnp.sum(s * wa2_ref[2:3, :], axis=1, keepdims=True) + ba2_ref[0, 2]
        m = jnp.maximum(jnp.maximum(a0, a1), a2)
        e0 = jnp.exp(a0 - m)
        e1 = jnp.exp(a1 - m)
        e2 = jnp.exp(a2 - m)
        out_ref[...] = (e0 * hxm + e1 * hum + e2 * hz) / (e0 + e1 + e2)


def _tc_dense(x_g, u_g, aggx, aggu, batch3, s_g, *weights):
    full = lambda shape: pl.BlockSpec(shape, lambda i: (0,) * len(shape))
    blk = pl.BlockSpec((BLK, D), lambda i: (i, 0))
    in_specs = [
        blk, blk, blk, blk,
        pl.BlockSpec((1, 1, BLK), lambda i: (i, 0, 0)),
        full((B, D)),
        full((D, D)), full((1, D)), full((D, D)), full((1, D)),
        full((1, 1)), full((1, D)), full((1, D)), full((1, D)), full((1, D)),
        full((D, D)), full((1, D)), full((D, D)), full((1, D)),
        full((1, 1)), full((1, D)), full((1, D)), full((1, D)), full((1, D)),
        full((D, D)), full((1, D)),
        full((D, D)), full((D, D)), full((D, D)), full((1, D)),
        full((8, D)), full((1, D)),
    ]
    return pl.pallas_call(
        _tc_body,
        grid=(NBLK,),
        in_specs=in_specs,
        out_specs=pl.BlockSpec((B, D), lambda i: (0, 0)),
        out_shape=jax.ShapeDtypeStruct((B, D), jnp.float32),
        scratch_shapes=[
            pltpu.VMEM((B, D), jnp.float32),
            pltpu.VMEM((B, D), jnp.float32),
            pltpu.VMEM((B, D), jnp.float32),
        ],
        compiler_params=pltpu.CompilerParams(
            dimension_semantics=("arbitrary",)),
    )(x_g, u_g, aggx, aggu, batch3, s_g, *weights)


def kernel(A_g, X_g, U_g, S_g, batch,
           W1x, b1x, W2x, b2x, eps_x, gamma_x, beta_x,
           W1u, b1u, W2u, b2u, eps_u, gamma_u, beta_u,
           Wz, bz, Wa1, ba1, Wa2, ba2,
           noise_gx, noise_bx, noise_gu, noise_bu):
    npad_e = EPAD - E
    pad_src = (jnp.arange(npad_e, dtype=jnp.int32) % 128)
    pad_dst = N + (jnp.arange(npad_e, dtype=jnp.int32) % (NPAD - N))
    tail = jnp.zeros((GRP * CHUNK,), jnp.int32)
    src2d = jnp.concatenate([A_g[0].astype(jnp.int32), pad_src, tail]).reshape(-1, CHUNK)
    dst2d = jnp.concatenate([A_g[1].astype(jnp.int32), pad_dst, tail]).reshape(-1, CHUNK)
    zinit = jnp.zeros((NPAD, D), jnp.float32)

    aggx, aggu = _sc_aggregate(src2d, dst2d, X_g, U_g, zinit)

    batch3 = batch.astype(jnp.int32).reshape(NBLK, 1, BLK)
    r = lambda a: a.reshape(1, D)
    weights = (
        W1x, r(b1x), W2x, r(b2x), eps_x.reshape(1, 1), r(gamma_x), r(beta_x),
        noise_gx, noise_bx,
        W1u, r(b1u), W2u, r(b2u), eps_u.reshape(1, 1), r(gamma_u), r(beta_u),
        noise_gu, noise_bu,
        Wz, r(bz),
        Wa1[:, 0:D], Wa1[:, D:2 * D], Wa1[:, 2 * D:3 * D], r(ba1),
        jnp.zeros((8, D), jnp.float32).at[0:3].set(Wa2),
        jnp.zeros((1, D), jnp.float32).at[0, 0:3].set(ba2),
    )
    return _tc_dense(X_g, U_g, aggx[:N], aggu[:N], batch3, S_g, *weights)


# speedup vs baseline: 8.3026x; 1.0181x over previous
import functools

import jax
import jax.numpy as jnp
from jax import lax
from jax.experimental import pallas as pl
from jax.experimental.pallas import tpu as pltpu
from jax.experimental.pallas import tpu_sc as plsc

N = 10000
E = 320000
D = 128
B = 256

NPAD = 10240
STRIPE = NPAD // 16
CHUNK = 128
CH_PER_TILE = 160
GRP = 16
NGRP = CH_PER_TILE // GRP
EPT = CH_PER_TILE * CHUNK
EPAD = EPT * 16


def _sc_aggregate(src2d, dst2d, x_g, u_g, zinit):
    mesh = plsc.VectorSubcoreMesh(core_axis_name="c", subcore_axis_name="s")

    @functools.partial(
        pl.kernel,
        mesh=mesh,
        out_type=[jax.ShapeDtypeStruct((NPAD, D), jnp.float32),
                  jax.ShapeDtypeStruct((NPAD, D), jnp.float32)],
        scratch_types=[
            pltpu.VMEM((GRP, CHUNK), jnp.int32),
            pltpu.VMEM((GRP, CHUNK), jnp.int32),
            pltpu.VMEM((GRP, CHUNK), jnp.int32),
            pltpu.VMEM((GRP, CHUNK), jnp.int32),
            pltpu.VMEM((CHUNK, D), jnp.float32),
            pltpu.VMEM((CHUNK, D), jnp.float32),
            pltpu.VMEM_SHARED((NPAD, D), jnp.float32),
            pltpu.SemaphoreType.DMA,
            pltpu.SemaphoreType.DMA,
            pltpu.SemaphoreType.DMA,
            pltpu.SemaphoreType.DMA,
            pltpu.SemaphoreType.DMA,
            pltpu.SemaphoreType.DMA,
        ],
    )
    def sc_kernel(src_hbm, dst_hbm, x_hbm, u_hbm, z_hbm, outx_hbm, outu_hbm,
                  src_v0, src_v1, dst_v0, dst_v1, rows0, rows1, agg_sh,
                  sem_g0, sem_g1, sem_s0, sem_s1, sem_i0, sem_i1):
        cid = lax.axis_index("c")
        sid = lax.axis_index("s")

        def work(table_hbm, out_hbm):
            rows = (rows0, rows1)
            sem_g = (sem_g0, sem_g1)
            sem_s = (sem_s0, sem_s1)
            src_v = (src_v0, src_v1)
            dst_v = (dst_v0, dst_v1)
            sem_i = (sem_i0, sem_i1)

            def idx_load(g, s):
                row_base = sid * CH_PER_TILE + g * GRP
                a = pltpu.async_copy(src_hbm.at[pl.ds(row_base, GRP)],
                                     src_v[s], sem_i[s])
                b = pltpu.async_copy(dst_hbm.at[pl.ds(row_base, GRP)],
                                     dst_v[s], sem_i[s])
                return (a, b)

            def idx_wait_desc(g, s):
                row_base = sid * CH_PER_TILE + g * GRP
                a = pltpu.make_async_copy(src_hbm.at[pl.ds(row_base, GRP)],
                                          src_v[s], sem_i[s])
                b = pltpu.make_async_copy(dst_hbm.at[pl.ds(row_base, GRP)],
                                          dst_v[s], sem_i[s])
                return (a, b)

            idx_load(0, 0)
            pltpu.sync_copy(z_hbm.at[pl.ds(sid * STRIPE, STRIPE)],
                            agg_sh.at[pl.ds(sid * STRIPE, STRIPE)])
            plsc.subcore_barrier()

            def super_group(sg, carry):
                g0 = 2 * sg
                for dsc in idx_wait_desc(g0, 0):
                    dsc.wait()
                idx1 = idx_load(g0 + 1, 1)
                scat = [None, None]
                gath = [None, None]
                tot = 2 * GRP
                gath[0] = pltpu.async_copy(table_hbm.at[src_v[0].at[0]],
                                           rows[0], sem_g[0])
                for c in range(tot):
                    cur, nxt = c % 2, (c + 1) % 2
                    if c + 1 < tot:
                        if c + 1 == GRP:
                            for dsc in idx1:
                                dsc.wait()
                        if scat[nxt] is not None:
                            scat[nxt].wait()
                        s_n, r_n = divmod(c + 1, GRP)
                        gath[nxt] = pltpu.async_copy(
                            table_hbm.at[src_v[s_n].at[r_n]], rows[nxt],
                            sem_g[nxt])
                    gath[cur].wait()
                    s_c, r_c = divmod(c, GRP)
                    scat[cur] = pltpu.async_copy(
                        rows[cur], agg_sh.at[dst_v[s_c].at[r_c]], sem_s[cur],
                        add=True)
                    if c == GRP + 1:
                        idx_load(g0 + 2, 0)
                scat[0].wait()
                scat[1].wait()
                return carry

            lax.fori_loop(0, NGRP // 2, super_group, 0)
            for dsc in idx_wait_desc(NGRP, 0):
                dsc.wait()
            plsc.subcore_barrier()
            pltpu.sync_copy(agg_sh.at[pl.ds(sid * STRIPE, STRIPE)],
                            out_hbm.at[pl.ds(sid * STRIPE, STRIPE)])

        @pl.when(cid == 0)
        def _():
            work(x_hbm, outx_hbm)

        @pl.when(cid == 1)
        def _():
            work(u_hbm, outu_hbm)

    return sc_kernel(src2d, dst2d, x_g, u_g, zinit)


def _softplus(x):
    return jnp.logaddexp(x, 0.0)


_DN = (((1,), (1,)), ((), ()))
_HI = lax.Precision.HIGHEST

NBLK = 10
BLK = N // NBLK


def _tc_body(x_ref, u_ref, ax_ref, au_ref, b3_ref, sg_ref,
             w1x_ref, b1x_ref, w2x_ref, b2x_ref, epsx_ref, gx_ref, btx_ref, ngx_ref, nbx_ref,
             w1u_ref, b1u_ref, w2u_ref, b2u_ref, epsu_ref, gu_ref, btu_ref, ngu_ref, nbu_ref,
             wz_ref, bz_ref, wa1x_ref, wa1u_ref, wa1z_ref, ba1_ref, wa2_ref, ba2_ref,
             out_ref, accx, accu, cnt):
    i = pl.program_id(0)

    def encoder(xb, ab, w1, b1, w2, b2, eps, gam, bet, ng, nb):
        h0 = (1.0 + eps) * xb + ab
        z = lax.dot_general(h0, w1, _DN, precision=_HI,
                            preferred_element_type=jnp.float32) + b1
        z = jnp.maximum(z, 0.0)
        z = lax.dot_general(z, w2, _DN, precision=_HI,
                            preferred_element_type=jnp.float32) + b2
        gs = 1.0 + _softplus(gam) * ng
        bs = _softplus(bet) * nb
        z = gs * z + bs
        return z * jax.nn.sigmoid(z)

    hx = encoder(x_ref[...], ax_ref[...], w1x_ref[...], b1x_ref[...],
                 w2x_ref[...], b2x_ref[...], epsx_ref[0, 0], gx_ref[...],
                 btx_ref[...], ngx_ref[...], nbx_ref[...])
    hu = encoder(u_ref[...], au_ref[...], w1u_ref[...], b1u_ref[...],
                 w2u_ref[...], b2u_ref[...], epsu_ref[0, 0], gu_ref[...],
                 btu_ref[...], ngu_ref[...], nbu_ref[...])

    row = jnp.reshape(b3_ref[...], (1, BLK))
    iot = lax.broadcasted_iota(jnp.int32, (B, BLK), 0)
    oh = (iot == row).astype(jnp.float32)
    px = lax.dot_general(oh, hx, (((1,), (0,)), ((), ())), precision=_HI,
                         preferred_element_type=jnp.float32)
    pu = lax.dot_general(oh, hu, (((1,), (0,)), ((), ())), precision=_HI,
                         preferred_element_type=jnp.float32)
    pc = lax.dot_general(oh, jnp.ones((BLK, D), jnp.float32),
                         (((1,), (0,)), ((), ())), precision=_HI,
                         preferred_element_type=jnp.float32)

    @pl.when(i == 0)
    def _():
        accx[...] = px
        accu[...] = pu
        cnt[...] = pc

    @pl.when(i > 0)
    def _():
        accx[...] += px
        accu[...] += pu
        cnt[...] += pc

    @pl.when(i == NBLK - 1)
    def _():
        c = jnp.maximum(cnt[...], 1.0)
        hxm = accx[...] / c
        hum = accu[...] / c
        hz = lax.dot_general(sg_ref[...], wz_ref[...], _DN, precision=_HI,
                             preferred_element_type=jnp.float32) + bz_ref[...]
        s = (lax.dot_general(hxm, wa1x_ref[...], _DN, precision=_HI,
                             preferred_element_type=jnp.float32)
             + lax.dot_general(hum, wa1u_ref[...], _DN, precision=_HI,
                               preferred_element_type=jnp.float32)
             + lax.dot_general(hz, wa1z_ref[...], _DN, precision=_HI,
                               preferred_element_type=jnp.float32)
             + ba1_ref[...])
        s = jnp.maximum(s, 0.0)
        a0 = jnp.sum(s * wa2_ref[0:1, :], axis=1, keepdims=True) + ba2_ref[0, 0]
        a1 = jnp.sum(s * wa2_ref[1:2, :], axis=1, keepdims=True) + ba2_ref[0, 1]
        a2 = jnp.sum(s * wa2_ref[2:3, :], axis=1, keepdims=True) + ba2_ref[0, 2]
        m = jnp.maximum(jnp.maximum(a0, a1), a2)
        e0 = jnp.exp(a0 - m)
        e1 = jnp.exp(a1 - m)
        e2 = jnp.exp(a2 - m)
        out_ref[...] = (e0 * hxm + e1 * hum + e2 * hz) / (e0 + e1 + e2)


def _tc_dense(x_g, u_g, aggx, aggu, batch3, s_g, *weights):
    full = lambda shape: pl.BlockSpec(shape, lambda i: (0,) * len(shape))
    blk = pl.BlockSpec((BLK, D), lambda i: (i, 0))
    in_specs = [
        blk, blk, blk, blk,
        pl.BlockSpec((1, 1, BLK), lambda i: (i, 0, 0)),
        full((B, D)),
        full((D, D)), full((1, D)), full((D, D)), full((1, D)),
        full((1, 1)), full((1, D)), full((1, D)), full((1, D)), full((1, D)),
        full((D, D)), full((1, D)), full((D, D)), full((1, D)),
        full((1, 1)), full((1, D)), full((1, D)), full((1, D)), full((1, D)),
        full((D, D)), full((1, D)),
        full((D, D)), full((D, D)), full((D, D)), full((1, D)),
        full((8, D)), full((1, D)),
    ]
    return pl.pallas_call(
        _tc_body,
        grid=(NBLK,),
        in_specs=in_specs,
        out_specs=pl.BlockSpec((B, D), lambda i: (0, 0)),
        out_shape=jax.ShapeDtypeStruct((B, D), jnp.float32),
        scratch_shapes=[
            pltpu.VMEM((B, D), jnp.float32),
            pltpu.VMEM((B, D), jnp.float32),
            pltpu.VMEM((B, D), jnp.float32),
        ],
        compiler_params=pltpu.CompilerParams(
            dimension_semantics=("arbitrary",)),
    )(x_g, u_g, aggx, aggu, batch3, s_g, *weights)


def kernel(A_g, X_g, U_g, S_g, batch,
           W1x, b1x, W2x, b2x, eps_x, gamma_x, beta_x,
           W1u, b1u, W2u, b2u, eps_u, gamma_u, beta_u,
           Wz, bz, Wa1, ba1, Wa2, ba2,
           noise_gx, noise_bx, noise_gu, noise_bu):
    npad_e = EPAD - E
    pad_src = (jnp.arange(npad_e, dtype=jnp.int32) % 128)
    pad_dst = N + (jnp.arange(npad_e, dtype=jnp.int32) % (NPAD - N))
    tail = jnp.zeros((GRP * CHUNK,), jnp.int32)
    src2d = jnp.concatenate([A_g[0].astype(jnp.int32), pad_src, tail]).reshape(-1, CHUNK)
    dst2d = jnp.concatenate([A_g[1].astype(jnp.int32), pad_dst, tail]).reshape(-1, CHUNK)
    zinit = jnp.zeros((NPAD, D), jnp.float32)

    aggx, aggu = _sc_aggregate(src2d, dst2d, X_g, U_g, zinit)

    batch3 = batch.astype(jnp.int32).reshape(NBLK, 1, BLK)
    r = lambda a: a.reshape(1, D)
    weights = (
        W1x, r(b1x), W2x, r(b2x), eps_x.reshape(1, 1), r(gamma_x), r(beta_x),
        noise_gx, noise_bx,
        W1u, r(b1u), W2u, r(b2u), eps_u.reshape(1, 1), r(gamma_u), r(beta_u),
        noise_gu, noise_bu,
        Wz, r(bz),
        Wa1[:, 0:D], Wa1[:, D:2 * D], Wa1[:, 2 * D:3 * D], r(ba1),
        jnp.zeros((8, D), jnp.float32).at[0:3].set(Wa2),
        jnp.zeros((1, D), jnp.float32).at[0, 0:3].set(ba2),
    )
    return _tc_dense(X_g, U_g, aggx[:N], aggu[:N], batch3, S_g, *weights)


# speedup vs baseline: 10.0971x; 1.2161x over previous
import functools

import jax
import jax.numpy as jnp
from jax import lax
from jax.experimental import pallas as pl
from jax.experimental.pallas import tpu as pltpu
from jax.experimental.pallas import tpu_sc as plsc

N = 10000
E = 320000
D = 128
B = 256

NPAD = 10240
STRIPE = NPAD // 16
CHUNK = 128
CH_PER_TILE = 160
GRP = 16
NGRP = CH_PER_TILE // GRP
EPT = CH_PER_TILE * CHUNK
EPAD = EPT * 16


def _sc_aggregate(src2d, dst2d, x_g, u_g, zinit):
    mesh = plsc.VectorSubcoreMesh(core_axis_name="c", subcore_axis_name="s")

    @functools.partial(
        pl.kernel,
        mesh=mesh,
        out_type=[jax.ShapeDtypeStruct((NPAD, D), jnp.float32),
                  jax.ShapeDtypeStruct((NPAD, D), jnp.float32)],
        scratch_types=[
            pltpu.VMEM((GRP, CHUNK), jnp.int32),
            pltpu.VMEM((GRP, CHUNK), jnp.int32),
            pltpu.VMEM((GRP, CHUNK), jnp.int32),
            pltpu.VMEM((GRP, CHUNK), jnp.int32),
            pltpu.VMEM((CHUNK, D), jnp.float32),
            pltpu.VMEM((CHUNK, D), jnp.float32),
            pltpu.VMEM_SHARED((NPAD, D), jnp.float32),
            pltpu.SemaphoreType.DMA,
            pltpu.SemaphoreType.DMA,
            pltpu.SemaphoreType.DMA,
            pltpu.SemaphoreType.DMA,
            pltpu.SemaphoreType.DMA,
            pltpu.SemaphoreType.DMA,
        ],
    )
    def sc_kernel(src_hbm, dst_hbm, x_hbm, u_hbm, z_hbm, outx_hbm, outu_hbm,
                  src_v0, src_v1, dst_v0, dst_v1, rows0, rows1, agg_sh,
                  sem_g0, sem_g1, sem_s0, sem_s1, sem_i0, sem_i1):
        cid = lax.axis_index("c")
        sid = lax.axis_index("s")

        def work(table_hbm, out_hbm):
            rows = (rows0, rows1)
            sem_g = (sem_g0, sem_g1)
            sem_s = (sem_s0, sem_s1)
            src_v = (src_v0, src_v1)
            dst_v = (dst_v0, dst_v1)
            sem_i = (sem_i0, sem_i1)

            def idx_load(g, s):
                row_base = sid * CH_PER_TILE + g * GRP
                a = pltpu.async_copy(src_hbm.at[pl.ds(row_base, GRP)],
                                     src_v[s], sem_i[s])
                b = pltpu.async_copy(dst_hbm.at[pl.ds(row_base, GRP)],
                                     dst_v[s], sem_i[s])
                return (a, b)

            def idx_wait_desc(g, s):
                row_base = sid * CH_PER_TILE + g * GRP
                a = pltpu.make_async_copy(src_hbm.at[pl.ds(row_base, GRP)],
                                          src_v[s], sem_i[s])
                b = pltpu.make_async_copy(dst_hbm.at[pl.ds(row_base, GRP)],
                                          dst_v[s], sem_i[s])
                return (a, b)

            idx_load(0, 0)
            pltpu.sync_copy(z_hbm, agg_sh.at[pl.ds(sid * STRIPE, STRIPE)])
            plsc.subcore_barrier()

            def super_group(sg, carry):
                g0 = 2 * sg
                for dsc in idx_wait_desc(g0, 0):
                    dsc.wait()
                idx1 = idx_load(g0 + 1, 1)
                scat = [None, None]
                gath = [None, None]
                tot = 2 * GRP
                gath[0] = pltpu.async_copy(table_hbm.at[src_v[0].at[0]],
                                           rows[0], sem_g[0])
                for c in range(tot):
                    cur, nxt = c % 2, (c + 1) % 2
                    if c + 1 < tot:
                        if c + 1 == GRP:
                            for dsc in idx1:
                                dsc.wait()
                        if scat[nxt] is not None:
                            scat[nxt].wait()
                        s_n, r_n = divmod(c + 1, GRP)
                        gath[nxt] = pltpu.async_copy(
                            table_hbm.at[src_v[s_n].at[r_n]], rows[nxt],
                            sem_g[nxt])
                    gath[cur].wait()
                    s_c, r_c = divmod(c, GRP)
                    scat[cur] = pltpu.async_copy(
                        rows[cur], agg_sh.at[dst_v[s_c].at[r_c]], sem_s[cur],
                        add=True)
                    if c == GRP + 1:
                        idx_load(g0 + 2, 0)
                scat[0].wait()
                scat[1].wait()
                return carry

            lax.fori_loop(0, NGRP // 2, super_group, 0)
            for dsc in idx_wait_desc(NGRP, 0):
                dsc.wait()
            plsc.subcore_barrier()
            pltpu.sync_copy(agg_sh.at[pl.ds(sid * STRIPE, STRIPE)],
                            out_hbm.at[pl.ds(sid * STRIPE, STRIPE)])

        @pl.when(cid == 0)
        def _():
            work(x_hbm, outx_hbm)

        @pl.when(cid == 1)
        def _():
            work(u_hbm, outu_hbm)

    return sc_kernel(src2d, dst2d, x_g, u_g, zinit)


def _softplus(x):
    return jnp.logaddexp(x, 0.0)


_DN = (((1,), (1,)), ((), ()))
_HI = lax.Precision.HIGHEST


def _mm16(a, b, dn):
    return lax.dot_general(a.astype(jnp.bfloat16), b.astype(jnp.bfloat16),
                           dn, preferred_element_type=jnp.float32)

NBLK = 10
BLK = N // NBLK


def _tc_body(x_ref, u_ref, ax_ref, au_ref, b3_ref, sg_ref,
             w1x_ref, b1x_ref, w2x_ref, b2x_ref, epsx_ref, gx_ref, btx_ref, ngx_ref, nbx_ref,
             w1u_ref, b1u_ref, w2u_ref, b2u_ref, epsu_ref, gu_ref, btu_ref, ngu_ref, nbu_ref,
             wz_ref, bz_ref, wa1x_ref, wa1u_ref, wa1z_ref, ba1_ref, wa2_ref, ba2_ref,
             out_ref, accx, accu, cnt):
    i = pl.program_id(0)

    def encoder(xb, ab, w1, b1, w2, b2, eps, gam, bet, ng, nb):
        h0 = (1.0 + eps) * xb + ab
        z = _mm16(h0, w1, _DN) + b1
        z = jnp.maximum(z, 0.0)
        z = _mm16(z, w2, _DN) + b2
        gs = 1.0 + _softplus(gam) * ng
        bs = _softplus(bet) * nb
        z = gs * z + bs
        return z * jax.nn.sigmoid(z)

    hx = encoder(x_ref[...], ax_ref[...], w1x_ref[...], b1x_ref[...],
                 w2x_ref[...], b2x_ref[...], epsx_ref[0, 0], gx_ref[...],
                 btx_ref[...], ngx_ref[...], nbx_ref[...])
    hu = encoder(u_ref[...], au_ref[...], w1u_ref[...], b1u_ref[...],
                 w2u_ref[...], b2u_ref[...], epsu_ref[0, 0], gu_ref[...],
                 btu_ref[...], ngu_ref[...], nbu_ref[...])

    row = jnp.reshape(b3_ref[...], (1, BLK))
    iot = lax.broadcasted_iota(jnp.int32, (B, BLK), 0)
    oh = (iot == row).astype(jnp.bfloat16)
    dn_p = (((1,), (0,)), ((), ()))
    px = _mm16(oh, hx, dn_p)
    pu = _mm16(oh, hu, dn_p)
    pc = _mm16(oh, jnp.ones((BLK, D), jnp.bfloat16), dn_p)

    @pl.when(i == 0)
    def _():
        accx[...] = px
        accu[...] = pu
        cnt[...] = pc

    @pl.when(i > 0)
    def _():
        accx[...] += px
        accu[...] += pu
        cnt[...] += pc

    @pl.when(i == NBLK - 1)
    def _():
        c = jnp.maximum(cnt[...], 1.0)
        hxm = accx[...] / c
        hum = accu[...] / c
        hz = lax.dot_general(sg_ref[...], wz_ref[...], _DN, precision=_HI,
                             preferred_element_type=jnp.float32) + bz_ref[...]
        s = (lax.dot_general(hxm, wa1x_ref[...], _DN, precision=_HI,
                             preferred_element_type=jnp.float32)
             + lax.dot_general(hum, wa1u_ref[...], _DN, precision=_HI,
                               preferred_element_type=jnp.float32)
             + lax.dot_general(hz, wa1z_ref[...], _DN, precision=_HI,
                               preferred_element_type=jnp.float32)
             + ba1_ref[...])
        s = jnp.maximum(s, 0.0)
        a0 = jnp.sum(s * wa2_ref[0:1, :], axis=1, keepdims=True) + ba2_ref[0, 0]
        a1 = jnp.sum(s * wa2_ref[1:2, :], axis=1, keepdims=True) + ba2_ref[0, 1]
        a2 = jnp.sum(s * wa2_ref[2:3, :], axis=1, keepdims=True) + ba2_ref[0, 2]
        m = jnp.maximum(jnp.maximum(a0, a1), a2)
        e0 = jnp.exp(a0 - m)
        e1 = jnp.exp(a1 - m)
        e2 = jnp.exp(a2 - m)
        out_ref[...] = (e0 * hxm + e1 * hum + e2 * hz) / (e0 + e1 + e2)


def _tc_dense(x_g, u_g, aggx, aggu, batch3, s_g, *weights):
    full = lambda shape: pl.BlockSpec(shape, lambda i: (0,) * len(shape))
    blk = pl.BlockSpec((BLK, D), lambda i: (i, 0))
    in_specs = [
        blk, blk, blk, blk,
        pl.BlockSpec((1, 1, BLK), lambda i: (i, 0, 0)),
        full((B, D)),
        full((D, D)), full((1, D)), full((D, D)), full((1, D)),
        full((1, 1)), full((1, D)), full((1, D)), full((1, D)), full((1, D)),
        full((D, D)), full((1, D)), full((D, D)), full((1, D)),
        full((1, 1)), full((1, D)), full((1, D)), full((1, D)), full((1, D)),
        full((D, D)), full((1, D)),
        full((D, D)), full((D, D)), full((D, D)), full((1, D)),
        full((8, D)), full((1, D)),
    ]
    return pl.pallas_call(
        _tc_body,
        grid=(NBLK,),
        in_specs=in_specs,
        out_specs=pl.BlockSpec((B, D), lambda i: (0, 0)),
        out_shape=jax.ShapeDtypeStruct((B, D), jnp.float32),
        scratch_shapes=[
            pltpu.VMEM((B, D), jnp.float32),
            pltpu.VMEM((B, D), jnp.float32),
            pltpu.VMEM((B, D), jnp.float32),
        ],
        compiler_params=pltpu.CompilerParams(
            dimension_semantics=("arbitrary",)),
    )(x_g, u_g, aggx, aggu, batch3, s_g, *weights)


def kernel(A_g, X_g, U_g, S_g, batch,
           W1x, b1x, W2x, b2x, eps_x, gamma_x, beta_x,
           W1u, b1u, W2u, b2u, eps_u, gamma_u, beta_u,
           Wz, bz, Wa1, ba1, Wa2, ba2,
           noise_gx, noise_bx, noise_gu, noise_bu):
    npad_e = EPAD - E
    pad_src = (jnp.arange(npad_e, dtype=jnp.int32) % 128)
    pad_dst = N + (jnp.arange(npad_e, dtype=jnp.int32) % (NPAD - N))
    tail = jnp.zeros((GRP * CHUNK,), jnp.int32)
    src2d = jnp.concatenate([A_g[0].astype(jnp.int32), pad_src, tail]).reshape(-1, CHUNK)
    dst2d = jnp.concatenate([A_g[1].astype(jnp.int32), pad_dst, tail]).reshape(-1, CHUNK)
    zinit = jnp.zeros((STRIPE, D), jnp.float32)

    aggx, aggu = _sc_aggregate(src2d, dst2d, X_g, U_g, zinit)

    batch3 = batch.astype(jnp.int32).reshape(NBLK, 1, BLK)
    r = lambda a: a.reshape(1, D)
    weights = (
        W1x, r(b1x), W2x, r(b2x), eps_x.reshape(1, 1), r(gamma_x), r(beta_x),
        noise_gx, noise_bx,
        W1u, r(b1u), W2u, r(b2u), eps_u.reshape(1, 1), r(gamma_u), r(beta_u),
        noise_gu, noise_bu,
        Wz, r(bz),
        Wa1[:, 0:D], Wa1[:, D:2 * D], Wa1[:, 2 * D:3 * D], r(ba1),
        jnp.zeros((8, D), jnp.float32).at[0:3].set(Wa2),
        jnp.zeros((1, D), jnp.float32).at[0, 0:3].set(ba2),
    )
    return _tc_dense(X_g, U_g, aggx, aggu, batch3, S_g, *weights)


# speedup vs baseline: 10.2711x; 1.0172x over previous
import functools

import jax
import jax.numpy as jnp
from jax import lax
from jax.experimental import pallas as pl
from jax.experimental.pallas import tpu as pltpu
from jax.experimental.pallas import tpu_sc as plsc

N = 10000
E = 320000
D = 128
B = 256

NPAD = 10240
STRIPE = NPAD // 16
CHUNK = 128
CH_PER_TILE = 160
GRP = 16
NGRP = CH_PER_TILE // GRP
EPT = CH_PER_TILE * CHUNK
EPAD = EPT * 16


def _sc_aggregate(src2d, dst2d, x_g, u_g, zinit):
    mesh = plsc.VectorSubcoreMesh(core_axis_name="c", subcore_axis_name="s")

    @functools.partial(
        pl.kernel,
        mesh=mesh,
        out_type=[jax.ShapeDtypeStruct((NPAD, D), jnp.float32),
                  jax.ShapeDtypeStruct((NPAD, D), jnp.float32)],
        scratch_types=[
            pltpu.VMEM((GRP, CHUNK), jnp.int32),
            pltpu.VMEM((GRP, CHUNK), jnp.int32),
            pltpu.VMEM((GRP, CHUNK), jnp.int32),
            pltpu.VMEM((GRP, CHUNK), jnp.int32),
            pltpu.VMEM((CHUNK, D), jnp.float32),
            pltpu.VMEM((CHUNK, D), jnp.float32),
            pltpu.VMEM_SHARED((NPAD, D), jnp.float32),
            pltpu.SemaphoreType.DMA,
            pltpu.SemaphoreType.DMA,
            pltpu.SemaphoreType.DMA,
            pltpu.SemaphoreType.DMA,
            pltpu.SemaphoreType.DMA,
            pltpu.SemaphoreType.DMA,
        ],
    )
    def sc_kernel(src_hbm, dst_hbm, x_hbm, u_hbm, z_hbm, outx_hbm, outu_hbm,
                  src_v0, src_v1, dst_v0, dst_v1, rows0, rows1, agg_sh,
                  sem_g0, sem_g1, sem_s0, sem_s1, sem_i0, sem_i1):
        cid = lax.axis_index("c")
        sid = lax.axis_index("s")

        def work(table_hbm, out_hbm):
            rows = (rows0, rows1)
            sem_g = (sem_g0, sem_g1)
            sem_s = (sem_s0, sem_s1)
            src_v = (src_v0, src_v1)
            dst_v = (dst_v0, dst_v1)
            sem_i = (sem_i0, sem_i1)

            def idx_load(g, s):
                row_base = sid * CH_PER_TILE + g * GRP
                a = pltpu.async_copy(src_hbm.at[pl.ds(row_base, GRP)],
                                     src_v[s], sem_i[s])
                b = pltpu.async_copy(dst_hbm.at[pl.ds(row_base, GRP)],
                                     dst_v[s], sem_i[s])
                return (a, b)

            def idx_wait_desc(g, s):
                row_base = sid * CH_PER_TILE + g * GRP
                a = pltpu.make_async_copy(src_hbm.at[pl.ds(row_base, GRP)],
                                          src_v[s], sem_i[s])
                b = pltpu.make_async_copy(dst_hbm.at[pl.ds(row_base, GRP)],
                                          dst_v[s], sem_i[s])
                return (a, b)

            idx_load(0, 0)
            pltpu.sync_copy(z_hbm, agg_sh.at[pl.ds(sid * STRIPE, STRIPE)])
            plsc.subcore_barrier()

            def super_group(sg, carry):
                g0 = 2 * sg
                for dsc in idx_wait_desc(g0, 0):
                    dsc.wait()
                idx1 = idx_load(g0 + 1, 1)
                scat = [None, None]
                gath = [None, None]
                tot = 2 * GRP
                gath[0] = pltpu.async_copy(table_hbm.at[src_v[0].at[0]],
                                           rows[0], sem_g[0])
                for c in range(tot):
                    cur, nxt = c % 2, (c + 1) % 2
                    if c + 1 < tot:
                        if c + 1 == GRP:
                            for dsc in idx1:
                                dsc.wait()
                        if scat[nxt] is not None:
                            scat[nxt].wait()
                        s_n, r_n = divmod(c + 1, GRP)
                        gath[nxt] = pltpu.async_copy(
                            table_hbm.at[src_v[s_n].at[r_n]], rows[nxt],
                            sem_g[nxt])
                    gath[cur].wait()
                    s_c, r_c = divmod(c, GRP)
                    scat[cur] = pltpu.async_copy(
                        rows[cur], agg_sh.at[dst_v[s_c].at[r_c]], sem_s[cur],
                        add=True)
                    if c == GRP + 1:
                        idx_load(g0 + 2, 0)
                scat[0].wait()
                scat[1].wait()
                return carry

            lax.fori_loop(0, NGRP // 2, super_group, 0)
            for dsc in idx_wait_desc(NGRP, 0):
                dsc.wait()
            plsc.subcore_barrier()
            pltpu.sync_copy(agg_sh.at[pl.ds(sid * STRIPE, STRIPE)],
                            out_hbm.at[pl.ds(sid * STRIPE, STRIPE)])

        @pl.when(cid == 0)
        def _():
            work(x_hbm, outx_hbm)

        @pl.when(cid == 1)
        def _():
            work(u_hbm, outu_hbm)

    return sc_kernel(src2d, dst2d, x_g, u_g, zinit)


def _softplus(x):
    return jnp.logaddexp(x, 0.0)


_DN = (((1,), (1,)), ((), ()))
_HI = lax.Precision.HIGHEST


def _mm16(a, b, dn):
    return lax.dot_general(a.astype(jnp.bfloat16), b.astype(jnp.bfloat16),
                           dn, preferred_element_type=jnp.float32)

NBLK = 10
BLK = N // NBLK


def _tc_body(x_ref, u_ref, ax_ref, au_ref, b3_ref, sg_ref,
             w1x_ref, b1x_ref, w2x_ref, b2x_ref, epsx_ref, gx_ref, btx_ref, ngx_ref, nbx_ref,
             w1u_ref, b1u_ref, w2u_ref, b2u_ref, epsu_ref, gu_ref, btu_ref, ngu_ref, nbu_ref,
             wz_ref, bz_ref, wa1x_ref, wa1u_ref, wa1z_ref, ba1_ref, wa2_ref, ba2_ref,
             out_ref, accx, accu, cnt):
    i = pl.program_id(0)

    def encoder(xb, ab, w1, b1, w2, b2, eps, gam, bet, ng, nb):
        h0 = (1.0 + eps) * xb + ab
        z = _mm16(h0, w1, _DN) + b1
        z = jnp.maximum(z, 0.0)
        z = _mm16(z, w2, _DN) + b2
        gs = 1.0 + _softplus(gam) * ng
        bs = _softplus(bet) * nb
        z = gs * z + bs
        return z * jax.nn.sigmoid(z)

    hx = encoder(x_ref[...], ax_ref[...], w1x_ref[...], b1x_ref[...],
                 w2x_ref[...], b2x_ref[...], epsx_ref[0, 0], gx_ref[...],
                 btx_ref[...], ngx_ref[...], nbx_ref[...])
    hu = encoder(u_ref[...], au_ref[...], w1u_ref[...], b1u_ref[...],
                 w2u_ref[...], b2u_ref[...], epsu_ref[0, 0], gu_ref[...],
                 btu_ref[...], ngu_ref[...], nbu_ref[...])

    row = jnp.reshape(b3_ref[...], (1, BLK))
    iot = lax.broadcasted_iota(jnp.int32, (B, BLK), 0)
    oh = (iot == row).astype(jnp.bfloat16)
    dn_p = (((1,), (0,)), ((), ()))
    px = _mm16(oh, hx, dn_p)
    pu = _mm16(oh, hu, dn_p)
    pc = _mm16(oh, jnp.ones((BLK, D), jnp.bfloat16), dn_p)

    @pl.when(i == 0)
    def _():
        accx[...] = px
        accu[...] = pu
        cnt[...] = pc

    @pl.when(i > 0)
    def _():
        accx[...] += px
        accu[...] += pu
        cnt[...] += pc

    @pl.when(i == NBLK - 1)
    def _():
        c = jnp.maximum(cnt[...], 1.0)
        hxm = accx[...] / c
        hum = accu[...] / c
        hz = lax.dot_general(sg_ref[...], wz_ref[...], _DN, precision=_HI,
                             preferred_element_type=jnp.float32) + bz_ref[...]
        s = (lax.dot_general(hxm, wa1x_ref[...], _DN, precision=_HI,
                             preferred_element_type=jnp.float32)
             + lax.dot_general(hum, wa1u_ref[...], _DN, precision=_HI,
                               preferred_element_type=jnp.float32)
             + lax.dot_general(hz, wa1z_ref[...], _DN, precision=_HI,
                               preferred_element_type=jnp.float32)
             + ba1_ref[...])
        s = jnp.maximum(s, 0.0)
        a0 = jnp.sum(s * wa2_ref[0:1, :], axis=1, keepdims=True) + ba2_ref[0, 0]
        a1 = jnp.sum(s * wa2_ref[1:2, :], axis=1, keepdims=True) + ba2_ref[0, 1]
        a2 = jnp.sum(s * wa2_ref[2:3, :], axis=1, keepdims=True) + ba2_ref[0, 2]
        m = jnp.maximum(jnp.maximum(a0, a1), a2)
        e0 = jnp.exp(a0 - m)
        e1 = jnp.exp(a1 - m)
        e2 = jnp.exp(a2 - m)
        out_ref[...] = (e0 * hxm + e1 * hum + e2 * hz) / (e0 + e1 + e2)


def _tc_dense(x_g, u_g, aggx, aggu, batch3, s_g, *weights):
    full = lambda shape: pl.BlockSpec(shape, lambda i: (0,) * len(shape))
    blk = pl.BlockSpec((BLK, D), lambda i: (i, 0))
    in_specs = [
        blk, blk, blk, blk,
        pl.BlockSpec((1, 1, BLK), lambda i: (i, 0, 0)),
        full((B, D)),
        full((D, D)), full((1, D)), full((D, D)), full((1, D)),
        full((1, 1)), full((1, D)), full((1, D)), full((1, D)), full((1, D)),
        full((D, D)), full((1, D)), full((D, D)), full((1, D)),
        full((1, 1)), full((1, D)), full((1, D)), full((1, D)), full((1, D)),
        full((D, D)), full((1, D)),
        full((D, D)), full((D, D)), full((D, D)), full((1, D)),
        full((8, D)), full((1, D)),
    ]
    return pl.pallas_call(
        _tc_body,
        grid=(NBLK,),
        in_specs=in_specs,
        out_specs=pl.BlockSpec((B, D), lambda i: (0, 0)),
        out_shape=jax.ShapeDtypeStruct((B, D), jnp.float32),
        scratch_shapes=[
            pltpu.VMEM((B, D), jnp.float32),
            pltpu.VMEM((B, D), jnp.float32),
            pltpu.VMEM((B, D), jnp.float32),
        ],
        compiler_params=pltpu.CompilerParams(
            dimension_semantics=("arbitrary",)),
    )(x_g, u_g, aggx, aggu, batch3, s_g, *weights)


def kernel(A_g, X_g, U_g, S_g, batch,
           W1x, b1x, W2x, b2x, eps_x, gamma_x, beta_x,
           W1u, b1u, W2u, b2u, eps_u, gamma_u, beta_u,
           Wz, bz, Wa1, ba1, Wa2, ba2,
           noise_gx, noise_bx, noise_gu, noise_bu):
    npad_e = EPAD - E + GRP * CHUNK
    ar = jnp.arange(npad_e, dtype=jnp.int32)
    pads = jnp.stack([ar % 128, N + ar % (NPAD - N)])
    edges = jnp.concatenate([A_g.astype(jnp.int32), pads], axis=1)
    src2d = edges[0].reshape(-1, CHUNK)
    dst2d = edges[1].reshape(-1, CHUNK)
    zinit = jnp.zeros((STRIPE, D), jnp.float32)

    aggx, aggu = _sc_aggregate(src2d, dst2d, X_g, U_g, zinit)

    batch3 = batch.astype(jnp.int32).reshape(NBLK, 1, BLK)
    r = lambda a: a.reshape(1, D)
    weights = (
        W1x, r(b1x), W2x, r(b2x), eps_x.reshape(1, 1), r(gamma_x), r(beta_x),
        noise_gx, noise_bx,
        W1u, r(b1u), W2u, r(b2u), eps_u.reshape(1, 1), r(gamma_u), r(beta_u),
        noise_gu, noise_bu,
        Wz, r(bz),
        Wa1[:, 0:D], Wa1[:, D:2 * D], Wa1[:, 2 * D:3 * D], r(ba1),
        jnp.zeros((8, D), jnp.float32).at[0:3].set(Wa2),
        jnp.zeros((1, D), jnp.float32).at[0, 0:3].set(ba2),
    )
    return _tc_dense(X_g, U_g, aggx, aggu, batch3, S_g, *weights)


# speedup vs baseline: 10.2906x; 1.0019x over previous
import functools

import jax
import jax.numpy as jnp
from jax import lax
from jax.experimental import pallas as pl
from jax.experimental.pallas import tpu as pltpu
from jax.experimental.pallas import tpu_sc as plsc

N = 10000
E = 320000
D = 128
B = 256

NPAD = 10240
STRIPE = NPAD // 16
CHUNK = 128
CH_PER_TILE = 160
GRP = 16
NGRP = CH_PER_TILE // GRP
EPT = CH_PER_TILE * CHUNK
EPAD = EPT * 16


def _sc_aggregate(src2d, dst2d, x_g, u_g, zinit):
    mesh = plsc.VectorSubcoreMesh(core_axis_name="c", subcore_axis_name="s")

    @functools.partial(
        pl.kernel,
        mesh=mesh,
        out_type=[jax.ShapeDtypeStruct((NPAD, D), jnp.float32),
                  jax.ShapeDtypeStruct((NPAD, D), jnp.float32)],
        scratch_types=[
            pltpu.VMEM((GRP, CHUNK), jnp.int32),
            pltpu.VMEM((GRP, CHUNK), jnp.int32),
            pltpu.VMEM((GRP, CHUNK), jnp.int32),
            pltpu.VMEM((GRP, CHUNK), jnp.int32),
            pltpu.VMEM((CHUNK, D), jnp.float32),
            pltpu.VMEM((CHUNK, D), jnp.float32),
            pltpu.VMEM_SHARED((NPAD, D), jnp.float32),
            pltpu.SemaphoreType.DMA,
            pltpu.SemaphoreType.DMA,
            pltpu.SemaphoreType.DMA,
            pltpu.SemaphoreType.DMA,
            pltpu.SemaphoreType.DMA,
            pltpu.SemaphoreType.DMA,
        ],
    )
    def sc_kernel(src_hbm, dst_hbm, x_hbm, u_hbm, z_hbm, outx_hbm, outu_hbm,
                  src_v0, src_v1, dst_v0, dst_v1, rows0, rows1, agg_sh,
                  sem_g0, sem_g1, sem_s0, sem_s1, sem_i0, sem_i1):
        cid = lax.axis_index("c")
        sid = lax.axis_index("s")

        def work(table_hbm, out_hbm):
            rows = (rows0, rows1)
            sem_g = (sem_g0, sem_g1)
            sem_s = (sem_s0, sem_s1)
            src_v = (src_v0, src_v1)
            dst_v = (dst_v0, dst_v1)
            sem_i = (sem_i0, sem_i1)

            def idx_load(g, s):
                row_base = sid * CH_PER_TILE + g * GRP
                a = pltpu.async_copy(src_hbm.at[pl.ds(row_base, GRP)],
                                     src_v[s], sem_i[s])
                b = pltpu.async_copy(dst_hbm.at[pl.ds(row_base, GRP)],
                                     dst_v[s], sem_i[s])
                return (a, b)

            def idx_wait_desc(g, s):
                row_base = sid * CH_PER_TILE + g * GRP
                a = pltpu.make_async_copy(src_hbm.at[pl.ds(row_base, GRP)],
                                          src_v[s], sem_i[s])
                b = pltpu.make_async_copy(dst_hbm.at[pl.ds(row_base, GRP)],
                                          dst_v[s], sem_i[s])
                return (a, b)

            idx_load(0, 0)
            pltpu.sync_copy(z_hbm, agg_sh.at[pl.ds(sid * STRIPE, STRIPE)])
            plsc.subcore_barrier()

            def super_group(sg, carry):
                g0 = 2 * sg
                for dsc in idx_wait_desc(g0, 0):
                    dsc.wait()
                idx1 = idx_load(g0 + 1, 1)
                scat = [None, None]
                gath = [None, None]
                tot = 2 * GRP
                gath[0] = pltpu.async_copy(table_hbm.at[src_v[0].at[0]],
                                           rows[0], sem_g[0])
                for c in range(tot):
                    cur, nxt = c % 2, (c + 1) % 2
                    if c + 1 < tot:
                        if c + 1 == GRP:
                            for dsc in idx1:
                                dsc.wait()
                        if scat[nxt] is not None:
                            scat[nxt].wait()
                        s_n, r_n = divmod(c + 1, GRP)
                        gath[nxt] = pltpu.async_copy(
                            table_hbm.at[src_v[s_n].at[r_n]], rows[nxt],
                            sem_g[nxt])
                    gath[cur].wait()
                    s_c, r_c = divmod(c, GRP)
                    scat[cur] = pltpu.async_copy(
                        rows[cur], agg_sh.at[dst_v[s_c].at[r_c]], sem_s[cur],
                        add=True)
                    if c == GRP + 1:
                        idx_load(g0 + 2, 0)
                scat[0].wait()
                scat[1].wait()
                return carry

            lax.fori_loop(0, NGRP // 2, super_group, 0)
            for dsc in idx_wait_desc(NGRP, 0):
                dsc.wait()
            plsc.subcore_barrier()
            pltpu.sync_copy(agg_sh.at[pl.ds(sid * STRIPE, STRIPE)],
                            out_hbm.at[pl.ds(sid * STRIPE, STRIPE)])

        @pl.when(cid == 0)
        def _():
            work(x_hbm, outx_hbm)

        @pl.when(cid == 1)
        def _():
            work(u_hbm, outu_hbm)

    return sc_kernel(src2d, dst2d, x_g, u_g, zinit)


def _softplus(x):
    return jnp.logaddexp(x, 0.0)


_DN = (((1,), (1,)), ((), ()))
_HI = lax.Precision.HIGHEST


def _mm16(a, b, dn):
    return lax.dot_general(a.astype(jnp.bfloat16), b.astype(jnp.bfloat16),
                           dn, preferred_element_type=jnp.float32)

NBLK = 10
BLK = N // NBLK


def _tc_body(x_ref, u_ref, ax_ref, au_ref, b3_ref, sg_ref,
             w1x_ref, b1x_ref, w2x_ref, b2x_ref, epsx_ref, gx_ref, btx_ref, ngx_ref, nbx_ref,
             w1u_ref, b1u_ref, w2u_ref, b2u_ref, epsu_ref, gu_ref, btu_ref, ngu_ref, nbu_ref,
             wz_ref, bz_ref, wa1x_ref, wa1u_ref, wa1z_ref, ba1_ref, wa2_ref, ba2_ref,
             out_ref, accx, accu, cnt):
    i = pl.program_id(0)

    def encoder(xb, ab, w1, b1, w2, b2, eps, gam, bet, ng, nb):
        h0 = (1.0 + eps) * xb + ab
        z = _mm16(h0, w1, _DN) + b1
        z = jnp.maximum(z, 0.0)
        z = _mm16(z, w2, _DN) + b2
        gs = 1.0 + _softplus(gam) * ng
        bs = _softplus(bet) * nb
        z = gs * z + bs
        return z * jax.nn.sigmoid(z)

    hx = encoder(x_ref[...], ax_ref[...], w1x_ref[...], b1x_ref[...],
                 w2x_ref[...], b2x_ref[...], epsx_ref[0, 0], gx_ref[...],
                 btx_ref[...], ngx_ref[...], nbx_ref[...])
    hu = encoder(u_ref[...], au_ref[...], w1u_ref[...], b1u_ref[...],
                 w2u_ref[...], b2u_ref[...], epsu_ref[0, 0], gu_ref[...],
                 btu_ref[...], ngu_ref[...], nbu_ref[...])

    row = jnp.reshape(b3_ref[...], (1, BLK))
    iot = lax.broadcasted_iota(jnp.int32, (B, BLK), 0)
    oh = (iot == row).astype(jnp.bfloat16)
    dn_p = (((1,), (0,)), ((), ()))
    px = _mm16(oh, hx, dn_p)
    pu = _mm16(oh, hu, dn_p)
    pc = _mm16(oh, jnp.ones((BLK, D), jnp.bfloat16), dn_p)

    @pl.when(i == 0)
    def _():
        accx[...] = px
        accu[...] = pu
        cnt[...] = pc

    @pl.when(i > 0)
    def _():
        accx[...] += px
        accu[...] += pu
        cnt[...] += pc

    @pl.when(i == NBLK - 1)
    def _():
        c = jnp.maximum(cnt[...], 1.0)
        hxm = accx[...] / c
        hum = accu[...] / c
        hz = lax.dot_general(sg_ref[...], wz_ref[...], _DN, precision=_HI,
                             preferred_element_type=jnp.float32) + bz_ref[...]
        s = (lax.dot_general(hxm, wa1x_ref[...], _DN, precision=_HI,
                             preferred_element_type=jnp.float32)
             + lax.dot_general(hum, wa1u_ref[...], _DN, precision=_HI,
                               preferred_element_type=jnp.float32)
             + lax.dot_general(hz, wa1z_ref[...], _DN, precision=_HI,
                               preferred_element_type=jnp.float32)
             + ba1_ref[...])
        s = jnp.maximum(s, 0.0)
        a0 = jnp.sum(s * wa2_ref[0:1, :], axis=1, keepdims=True) + ba2_ref[0, 0]
        a1 = jnp.sum(s * wa2_ref[1:2, :], axis=1, keepdims=True) + ba2_ref[0, 1]
        a2 = jnp.sum(s * wa2_ref[2:3, :], axis=1, keepdims=True) + ba2_ref[0, 2]
        m = jnp.maximum(jnp.maximum(a0, a1), a2)
        e0 = jnp.exp(a0 - m)
        e1 = jnp.exp(a1 - m)
        e2 = jnp.exp(a2 - m)
        out_ref[...] = (e0 * hxm + e1 * hum + e2 * hz) / (e0 + e1 + e2)


def _tc_dense(x_g, u_g, aggx, aggu, batch3, s_g, *weights):
    full = lambda shape: pl.BlockSpec(shape, lambda i: (0,) * len(shape))
    blk = pl.BlockSpec((BLK, D), lambda i: (i, 0))
    in_specs = [
        blk, blk, blk, blk,
        pl.BlockSpec((1, 1, BLK), lambda i: (i, 0, 0)),
        full((B, D)),
        full((D, D)), full((1, D)), full((D, D)), full((1, D)),
        full((1, 1)), full((1, D)), full((1, D)), full((1, D)), full((1, D)),
        full((D, D)), full((1, D)), full((D, D)), full((1, D)),
        full((1, 1)), full((1, D)), full((1, D)), full((1, D)), full((1, D)),
        full((D, D)), full((1, D)),
        full((D, D)), full((D, D)), full((D, D)), full((1, D)),
        full((3, D)), full((1, 3)),
    ]
    return pl.pallas_call(
        _tc_body,
        grid=(NBLK,),
        in_specs=in_specs,
        out_specs=pl.BlockSpec((B, D), lambda i: (0, 0)),
        out_shape=jax.ShapeDtypeStruct((B, D), jnp.float32),
        scratch_shapes=[
            pltpu.VMEM((B, D), jnp.float32),
            pltpu.VMEM((B, D), jnp.float32),
            pltpu.VMEM((B, D), jnp.float32),
        ],
        compiler_params=pltpu.CompilerParams(
            dimension_semantics=("arbitrary",)),
    )(x_g, u_g, aggx, aggu, batch3, s_g, *weights)


def kernel(A_g, X_g, U_g, S_g, batch,
           W1x, b1x, W2x, b2x, eps_x, gamma_x, beta_x,
           W1u, b1u, W2u, b2u, eps_u, gamma_u, beta_u,
           Wz, bz, Wa1, ba1, Wa2, ba2,
           noise_gx, noise_bx, noise_gu, noise_bu):
    npad_e = EPAD - E + GRP * CHUNK
    ar = jnp.arange(npad_e, dtype=jnp.int32)
    pads = jnp.stack([ar % 128, N + ar % (NPAD - N)])
    edges = jnp.concatenate([A_g.astype(jnp.int32), pads], axis=1)
    src2d = edges[0].reshape(-1, CHUNK)
    dst2d = edges[1].reshape(-1, CHUNK)
    zinit = jnp.zeros((STRIPE, D), jnp.float32)

    aggx, aggu = _sc_aggregate(src2d, dst2d, X_g, U_g, zinit)

    batch3 = batch.astype(jnp.int32).reshape(NBLK, 1, BLK)
    r = lambda a: a.reshape(1, D)
    weights = (
        W1x, r(b1x), W2x, r(b2x), eps_x.reshape(1, 1), r(gamma_x), r(beta_x),
        noise_gx, noise_bx,
        W1u, r(b1u), W2u, r(b2u), eps_u.reshape(1, 1), r(gamma_u), r(beta_u),
        noise_gu, noise_bu,
        Wz, r(bz),
        Wa1[:, 0:D], Wa1[:, D:2 * D], Wa1[:, 2 * D:3 * D], r(ba1),
        Wa2, ba2.reshape(1, 3),
    )
    return _tc_dense(X_g, U_g, aggx, aggu, batch3, S_g, *weights)
